# initial kernel scaffold (unmeasured)
import functools

import jax
import jax.numpy as jnp
from jax import lax
from jax.experimental import pallas as pl
from jax.experimental.pallas import tpu as pltpu

N_DEV = 8
B_LOC = 2
SQ = 512
HQ_LOC = 8
DH = 64
D_MODEL = 768
D_SHARD = HQ_LOC * DH


def _attn_shard(jj, xbf_ref, kt_ref, vt_ref, wq_buf, wo_buf, ctx_ref, acc_ref,
                mask):
    wq_j = wq_buf[jj]
    qf = jnp.dot(xbf_ref[...], wq_j,
                 preferred_element_type=jnp.float32)
    qf = (qf * 0.125).astype(jnp.bfloat16)
    for b in range(B_LOC):
        for h in range(HQ_LOC):
            q = qf[b * SQ:(b + 1) * SQ, h * DH:(h + 1) * DH]
            kv_idx = (jj * HQ_LOC + h) * B_LOC + b
            k = kt_ref[kv_idx]
            s = lax.dot_general(q, k, (((1,), (1,)), ((), ())),
                                preferred_element_type=jnp.float32)
            s = jnp.where(mask, s, -1e9)
            mx = jnp.max(s, axis=1, keepdims=True)
            w = jnp.exp(s - mx)
            w = w / jnp.sum(w, axis=1, keepdims=True)
            ctx = jnp.dot(w.astype(jnp.bfloat16), vt_ref[kv_idx],
                          preferred_element_type=jnp.float32)
            ctx_ref[b * SQ:(b + 1) * SQ, h * DH:(h + 1) * DH] = (
                ctx.astype(jnp.bfloat16))
    acc_ref[...] += jnp.dot(ctx_ref[...], wo_buf[jj],
                            preferred_element_type=jnp.float32)


def kernel(x, Wq, K_ext, V_ext, Wo):
    i_idx = lax.axis_index("i")

    Ks = lax.dynamic_slice_in_dim(K_ext, i_idx * B_LOC, B_LOC, 0)
    Vs = lax.dynamic_slice_in_dim(V_ext, i_idx * B_LOC, B_LOC, 0)
    Kt = jnp.transpose(Ks, (2, 0, 1, 3)).astype(jnp.bfloat16)
    Kt = Kt.reshape(64 * B_LOC, SQ, DH)
    Vt = jnp.transpose(Vs, (2, 0, 1, 3)).astype(jnp.bfloat16)
    Vt = Vt.reshape(64 * B_LOC, SQ, DH)
    x2 = x.reshape(B_LOC * SQ, D_MODEL)
    Wq_bf = Wq.astype(jnp.bfloat16)
    Wo_bf = Wo.astype(jnp.bfloat16)

    def body(x_ref, wq_ref, wo_ref, kt_ref, vt_ref, out_ref,
             xbf, wq_buf, wo_buf, ctx_ref, acc_ref,
             send_q, send_o, recv_q, recv_o):
        my = lax.axis_index("i")
        left = lax.rem(my + N_DEV - 1, N_DEV)
        right = lax.rem(my + 1, N_DEV)

        barrier_sem = pltpu.get_barrier_semaphore()
        pl.semaphore_signal(barrier_sem, inc=1, device_id=(left,),
                            device_id_type=pl.DeviceIdType.MESH)
        pl.semaphore_signal(barrier_sem, inc=1, device_id=(right,),
                            device_id_type=pl.DeviceIdType.MESH)
        pl.semaphore_wait(barrier_sem, 2)

        xbf[...] = x_ref[...].astype(jnp.bfloat16)
        wq_buf[pl.ds(my, 1)] = wq_ref[...][None]
        wo_buf[pl.ds(my, 1)] = wo_ref[...][None]

        qb = lax.broadcasted_iota(jnp.int32, (SQ, SQ), 0) // 64
        kb = lax.broadcasted_iota(jnp.int32, (SQ, SQ), 1) // 64
        mask = (qb == kb) | (kb == 0) | (lax.rem(qb + kb, 3) == 0)

        acc_ref[...] = jnp.zeros_like(acc_ref)
        _attn_shard(my, xbf, kt_ref, vt_ref, wq_buf, wo_buf, ctx_ref,
                    acc_ref, mask)

        def hop(h, _):
            o_send = lax.rem(my + N_DEV - h, N_DEV)
            o_recv = lax.rem(my + N_DEV - 1 - h, N_DEV)
            sq = pltpu.make_async_remote_copy(
                src_ref=wq_buf.at[o_send], dst_ref=wq_buf.at[o_send],
                send_sem=send_q.at[h], recv_sem=recv_q.at[h],
                device_id=(right,), device_id_type=pl.DeviceIdType.MESH)
            so = pltpu.make_async_remote_copy(
                src_ref=wo_buf.at[o_send], dst_ref=wo_buf.at[o_send],
                send_sem=send_o.at[h], recv_sem=recv_o.at[h],
                device_id=(right,), device_id_type=pl.DeviceIdType.MESH)
            sq.start()
            so.start()
            rq = pltpu.make_async_remote_copy(
                src_ref=wq_buf.at[o_recv], dst_ref=wq_buf.at[o_recv],
                send_sem=send_q.at[h], recv_sem=recv_q.at[h],
                device_id=(left,), device_id_type=pl.DeviceIdType.MESH)
            ro = pltpu.make_async_remote_copy(
                src_ref=wo_buf.at[o_recv], dst_ref=wo_buf.at[o_recv],
                send_sem=send_o.at[h], recv_sem=recv_o.at[h],
                device_id=(left,), device_id_type=pl.DeviceIdType.MESH)
            rq.wait_recv()
            ro.wait_recv()
            sq.wait_send()
            so.wait_send()
            _attn_shard(o_recv, xbf, kt_ref, vt_ref, wq_buf, wo_buf,
                        ctx_ref, acc_ref, mask)
            return 0

        lax.fori_loop(0, N_DEV - 1, hop, 0)
        out_ref[...] = acc_ref[...]

    out = pl.pallas_call(
        body,
        out_shape=jax.ShapeDtypeStruct((B_LOC * SQ, D_MODEL), jnp.float32),
        in_specs=[pl.BlockSpec(memory_space=pltpu.VMEM)] * 5,
        out_specs=pl.BlockSpec(memory_space=pltpu.VMEM),
        scratch_shapes=[
            pltpu.VMEM((B_LOC * SQ, D_MODEL), jnp.bfloat16),
            pltpu.VMEM((N_DEV, D_MODEL, D_SHARD), jnp.bfloat16),
            pltpu.VMEM((N_DEV, D_SHARD, D_MODEL), jnp.bfloat16),
            pltpu.VMEM((B_LOC * SQ, D_SHARD), jnp.bfloat16),
            pltpu.VMEM((B_LOC * SQ, D_MODEL), jnp.float32),
            pltpu.SemaphoreType.DMA((N_DEV - 1,)),
            pltpu.SemaphoreType.DMA((N_DEV - 1,)),
            pltpu.SemaphoreType.DMA((N_DEV - 1,)),
            pltpu.SemaphoreType.DMA((N_DEV - 1,)),
        ],
        compiler_params=pltpu.CompilerParams(collective_id=0),
    )(x2, Wq_bf, Wo_bf, Kt, Vt)
    return out.reshape(B_LOC, SQ, D_MODEL)


# baseline (device time: 255151 ns/iter reference)
import jax
import jax.numpy as jnp
from jax import lax
from jax.experimental import pallas as pl
from jax.experimental.pallas import tpu as pltpu

N_DEV = 8
B_LOC = 2
SQ = 512
HQ_LOC = 8
DH = 64
D_MODEL = 768
D_SHARD = HQ_LOC * DH
KV_SLAB = HQ_LOC * B_LOC


def _attn_shard(jj, xbf_ref, kbuf, vbuf, wq_buf, wo_buf, ctx_ref, out_ref,
                mask):
    wq_j = wq_buf[jj]
    qf = jnp.dot(xbf_ref[...], wq_j,
                 preferred_element_type=jnp.float32)
    qf = (qf * 0.125).astype(jnp.bfloat16)
    for b in range(B_LOC):
        for h in range(HQ_LOC):
            q = qf[b * SQ:(b + 1) * SQ, h * DH:(h + 1) * DH]
            k = kbuf[h * B_LOC + b]
            s = lax.dot_general(q, k, (((1,), (1,)), ((), ())),
                                preferred_element_type=jnp.float32)
            s = jnp.where(mask, s, -1e9)
            mx = jnp.max(s, axis=1, keepdims=True)
            w = jnp.exp(s - mx)
            w = w / jnp.sum(w, axis=1, keepdims=True)
            ctx = jnp.dot(w.astype(jnp.bfloat16), vbuf[h * B_LOC + b],
                          preferred_element_type=jnp.float32)
            ctx_ref[b * SQ:(b + 1) * SQ, h * DH:(h + 1) * DH] = (
                ctx.astype(jnp.bfloat16))
    out_ref[...] += jnp.dot(ctx_ref[...], wo_buf[jj],
                            preferred_element_type=jnp.float32)


def kernel(x, Wq, K_ext, V_ext, Wo):
    i_idx = lax.axis_index("i")

    Ks = lax.dynamic_slice_in_dim(K_ext, i_idx * B_LOC, B_LOC, 0)
    Vs = lax.dynamic_slice_in_dim(V_ext, i_idx * B_LOC, B_LOC, 0)
    Kt = jnp.transpose(Ks, (2, 0, 1, 3)).astype(jnp.bfloat16)
    Kt = Kt.reshape(64 * B_LOC, SQ, DH)
    Vt = jnp.transpose(Vs, (2, 0, 1, 3)).astype(jnp.bfloat16)
    Vt = Vt.reshape(64 * B_LOC, SQ, DH)
    x2 = x.reshape(B_LOC * SQ, D_MODEL).astype(jnp.bfloat16)
    Wq_bf = Wq.astype(jnp.bfloat16)
    Wo_bf = Wo.astype(jnp.bfloat16)

    def body(x_ref, wq_ref, wo_ref, kt_hbm, vt_hbm, out_ref,
             wq_buf, wo_buf, ctx_ref, kbuf, vbuf,
             kv_sem, send_q, send_o, recv_q, recv_o):
        my = lax.axis_index("i")
        left = lax.rem(my + N_DEV - 1, N_DEV)
        right = lax.rem(my + 1, N_DEV)

        barrier_sem = pltpu.get_barrier_semaphore()
        pl.semaphore_signal(barrier_sem, inc=1, device_id=(left,),
                            device_id_type=pl.DeviceIdType.MESH)
        pl.semaphore_signal(barrier_sem, inc=1, device_id=(right,),
                            device_id_type=pl.DeviceIdType.MESH)
        pl.semaphore_wait(barrier_sem, 2)

        wq_buf[pl.ds(my, 1)] = wq_ref[...][None]
        wo_buf[pl.ds(my, 1)] = wo_ref[...][None]

        qb = lax.broadcasted_iota(jnp.int32, (SQ, SQ), 0) // 64
        kb = lax.broadcasted_iota(jnp.int32, (SQ, SQ), 1) // 64
        mask = (qb == kb) | (kb == 0) | (lax.rem(qb + kb, 3) == 0)

        out_ref[...] = jnp.zeros_like(out_ref)

        def load_kv(jj):
            ck = pltpu.make_async_copy(
                kt_hbm.at[pl.ds(jj * KV_SLAB, KV_SLAB)], kbuf, kv_sem.at[0])
            cv = pltpu.make_async_copy(
                vt_hbm.at[pl.ds(jj * KV_SLAB, KV_SLAB)], vbuf, kv_sem.at[1])
            ck.start()
            cv.start()
            ck.wait()
            cv.wait()

        load_kv(my)
        _attn_shard(my, x_ref, kbuf, vbuf, wq_buf, wo_buf, ctx_ref,
                    out_ref, mask)

        def hop(h, _):
            o_send = lax.rem(my + N_DEV - h, N_DEV)
            o_recv = lax.rem(my + N_DEV - 1 - h, N_DEV)
            sq = pltpu.make_async_remote_copy(
                src_ref=wq_buf.at[o_send], dst_ref=wq_buf.at[o_send],
                send_sem=send_q.at[h], recv_sem=recv_q.at[h],
                device_id=(right,), device_id_type=pl.DeviceIdType.MESH)
            so = pltpu.make_async_remote_copy(
                src_ref=wo_buf.at[o_send], dst_ref=wo_buf.at[o_send],
                send_sem=send_o.at[h], recv_sem=recv_o.at[h],
                device_id=(right,), device_id_type=pl.DeviceIdType.MESH)
            sq.start()
            so.start()
            load_kv(o_recv)
            rq = pltpu.make_async_remote_copy(
                src_ref=wq_buf.at[o_recv], dst_ref=wq_buf.at[o_recv],
                send_sem=send_q.at[h], recv_sem=recv_q.at[h],
                device_id=(left,), device_id_type=pl.DeviceIdType.MESH)
            ro = pltpu.make_async_remote_copy(
                src_ref=wo_buf.at[o_recv], dst_ref=wo_buf.at[o_recv],
                send_sem=send_o.at[h], recv_sem=recv_o.at[h],
                device_id=(left,), device_id_type=pl.DeviceIdType.MESH)
            rq.wait_recv()
            ro.wait_recv()
            sq.wait_send()
            so.wait_send()
            _attn_shard(o_recv, x_ref, kbuf, vbuf, wq_buf, wo_buf,
                        ctx_ref, out_ref, mask)
            return 0

        lax.fori_loop(0, N_DEV - 1, hop, 0)

    out = pl.pallas_call(
        body,
        out_shape=jax.ShapeDtypeStruct((B_LOC * SQ, D_MODEL), jnp.float32),
        in_specs=[
            pl.BlockSpec(memory_space=pltpu.VMEM),
            pl.BlockSpec(memory_space=pltpu.VMEM),
            pl.BlockSpec(memory_space=pltpu.VMEM),
            pl.BlockSpec(memory_space=pl.ANY),
            pl.BlockSpec(memory_space=pl.ANY),
        ],
        out_specs=pl.BlockSpec(memory_space=pltpu.VMEM),
        scratch_shapes=[
            pltpu.VMEM((N_DEV, D_MODEL, D_SHARD), jnp.bfloat16),
            pltpu.VMEM((N_DEV, D_SHARD, D_MODEL), jnp.bfloat16),
            pltpu.VMEM((B_LOC * SQ, D_SHARD), jnp.bfloat16),
            pltpu.VMEM((KV_SLAB, SQ, DH), jnp.bfloat16),
            pltpu.VMEM((KV_SLAB, SQ, DH), jnp.bfloat16),
            pltpu.SemaphoreType.DMA((2,)),
            pltpu.SemaphoreType.DMA((N_DEV - 1,)),
            pltpu.SemaphoreType.DMA((N_DEV - 1,)),
            pltpu.SemaphoreType.DMA((N_DEV - 1,)),
            pltpu.SemaphoreType.DMA((N_DEV - 1,)),
        ],
        compiler_params=pltpu.CompilerParams(collective_id=0),
    )(x2, Wq_bf, Wo_bf, Kt, Vt)
    return out.reshape(B_LOC, SQ, D_MODEL)


# device time: 184906 ns/iter; 1.3799x vs baseline; 1.3799x over previous
import jax
import jax.numpy as jnp
from jax import lax
from jax.experimental import pallas as pl
from jax.experimental.pallas import tpu as pltpu

N_DEV = 8
B_LOC = 2
SQ = 512
HQ_LOC = 8
DH = 64
D_MODEL = 768
D_SHARD = HQ_LOC * DH
KV_SLAB = HQ_LOC * B_LOC


def _attn_shard(jj, xbf_ref, kbuf, vbuf, wq_buf, wo_buf, ctx_ref, out_ref,
                mask):
    wq_j = wq_buf[jj]
    qf = jnp.dot(xbf_ref[...], wq_j,
                 preferred_element_type=jnp.float32)
    qf = (qf * 0.125).astype(jnp.bfloat16)
    for b in range(B_LOC):
        for h in range(HQ_LOC):
            q = qf[b * SQ:(b + 1) * SQ, h * DH:(h + 1) * DH]
            k = kbuf[h * B_LOC + b]
            s = lax.dot_general(q, k, (((1,), (1,)), ((), ())),
                                preferred_element_type=jnp.float32)
            s = jnp.where(mask, s, -1e9)
            mx = jnp.max(s, axis=1, keepdims=True)
            w = jnp.exp(s - mx)
            w = w / jnp.sum(w, axis=1, keepdims=True)
            ctx = jnp.dot(w.astype(jnp.bfloat16), vbuf[h * B_LOC + b],
                          preferred_element_type=jnp.float32)
            ctx_ref[b * SQ:(b + 1) * SQ, h * DH:(h + 1) * DH] = (
                ctx.astype(jnp.bfloat16))
    out_ref[...] += jnp.dot(ctx_ref[...], wo_buf[jj],
                            preferred_element_type=jnp.float32)


def kernel(x, Wq, K_ext, V_ext, Wo):
    i_idx = lax.axis_index("i")

    Ks = lax.dynamic_slice_in_dim(K_ext, i_idx * B_LOC, B_LOC, 0)
    Vs = lax.dynamic_slice_in_dim(V_ext, i_idx * B_LOC, B_LOC, 0)
    Kt = jnp.transpose(Ks, (2, 0, 1, 3)).astype(jnp.bfloat16)
    Kt = Kt.reshape(64 * B_LOC, SQ, DH)
    Vt = jnp.transpose(Vs, (2, 0, 1, 3)).astype(jnp.bfloat16)
    Vt = Vt.reshape(64 * B_LOC, SQ, DH)
    x2 = x.reshape(B_LOC * SQ, D_MODEL).astype(jnp.bfloat16)
    Wq_bf = Wq.astype(jnp.bfloat16)
    Wo_bf = Wo.astype(jnp.bfloat16)

    def body(x_ref, wq_ref, wo_ref, kt_hbm, vt_hbm, out_ref,
             wq_buf, wo_buf, ctx_ref, kbuf, vbuf,
             kv_sem, send_q, send_o, recv_q, recv_o):
        my = lax.axis_index("i")
        left = lax.rem(my + N_DEV - 1, N_DEV)
        right = lax.rem(my + 1, N_DEV)

        barrier_sem = pltpu.get_barrier_semaphore()
        pl.semaphore_signal(barrier_sem, inc=1, device_id=(left,),
                            device_id_type=pl.DeviceIdType.MESH)
        pl.semaphore_signal(barrier_sem, inc=1, device_id=(right,),
                            device_id_type=pl.DeviceIdType.MESH)
        pl.semaphore_wait(barrier_sem, 2)

        wq_buf[pl.ds(my, 1)] = wq_ref[...][None]
        wo_buf[pl.ds(my, 1)] = wo_ref[...][None]

        qb = lax.broadcasted_iota(jnp.int32, (SQ, SQ), 0) // 64
        kb = lax.broadcasted_iota(jnp.int32, (SQ, SQ), 1) // 64
        mask = (qb == kb) | (kb == 0) | (lax.rem(qb + kb, 3) == 0)

        out_ref[...] = jnp.zeros_like(out_ref)

        def send_pair(h, o_send):
            sq = pltpu.make_async_remote_copy(
                src_ref=wq_buf.at[o_send], dst_ref=wq_buf.at[o_send],
                send_sem=send_q.at[h], recv_sem=recv_q.at[h],
                device_id=(right,), device_id_type=pl.DeviceIdType.MESH)
            so = pltpu.make_async_remote_copy(
                src_ref=wo_buf.at[o_send], dst_ref=wo_buf.at[o_send],
                send_sem=send_o.at[h], recv_sem=recv_o.at[h],
                device_id=(right,), device_id_type=pl.DeviceIdType.MESH)
            sq.start()
            so.start()

        def start_kv(jj):
            pltpu.make_async_copy(
                kt_hbm.at[pl.ds(jj * KV_SLAB, KV_SLAB)], kbuf,
                kv_sem.at[0]).start()
            pltpu.make_async_copy(
                vt_hbm.at[pl.ds(jj * KV_SLAB, KV_SLAB)], vbuf,
                kv_sem.at[1]).start()

        def wait_kv():
            pltpu.make_async_copy(
                kt_hbm.at[pl.ds(0, KV_SLAB)], kbuf, kv_sem.at[0]).wait()
            pltpu.make_async_copy(
                vt_hbm.at[pl.ds(0, KV_SLAB)], vbuf, kv_sem.at[1]).wait()

        send_pair(0, my)
        start_kv(my)
        wait_kv()
        _attn_shard(my, x_ref, kbuf, vbuf, wq_buf, wo_buf, ctx_ref,
                    out_ref, mask)

        def hop(h, _):
            o_send = lax.rem(my + N_DEV - h, N_DEV)
            o_recv = lax.rem(my + N_DEV - 1 - h, N_DEV)
            rq = pltpu.make_async_remote_copy(
                src_ref=wq_buf.at[o_recv], dst_ref=wq_buf.at[o_recv],
                send_sem=send_q.at[h], recv_sem=recv_q.at[h],
                device_id=(left,), device_id_type=pl.DeviceIdType.MESH)
            ro = pltpu.make_async_remote_copy(
                src_ref=wo_buf.at[o_recv], dst_ref=wo_buf.at[o_recv],
                send_sem=send_o.at[h], recv_sem=recv_o.at[h],
                device_id=(left,), device_id_type=pl.DeviceIdType.MESH)
            rq.wait_recv()
            ro.wait_recv()

            @pl.when(h < N_DEV - 2)
            def _():
                send_pair(h + 1, o_recv)

            wsq = pltpu.make_async_remote_copy(
                src_ref=wq_buf.at[o_send], dst_ref=wq_buf.at[o_send],
                send_sem=send_q.at[h], recv_sem=recv_q.at[h],
                device_id=(right,), device_id_type=pl.DeviceIdType.MESH)
            wso = pltpu.make_async_remote_copy(
                src_ref=wo_buf.at[o_send], dst_ref=wo_buf.at[o_send],
                send_sem=send_o.at[h], recv_sem=recv_o.at[h],
                device_id=(right,), device_id_type=pl.DeviceIdType.MESH)
            wsq.wait_send()
            wso.wait_send()

            start_kv(o_recv)
            wait_kv()
            _attn_shard(o_recv, x_ref, kbuf, vbuf, wq_buf, wo_buf,
                        ctx_ref, out_ref, mask)
            return 0

        lax.fori_loop(0, N_DEV - 1, hop, 0)

    out = pl.pallas_call(
        body,
        out_shape=jax.ShapeDtypeStruct((B_LOC * SQ, D_MODEL), jnp.float32),
        in_specs=[
            pl.BlockSpec(memory_space=pltpu.VMEM),
            pl.BlockSpec(memory_space=pltpu.VMEM),
            pl.BlockSpec(memory_space=pltpu.VMEM),
            pl.BlockSpec(memory_space=pl.ANY),
            pl.BlockSpec(memory_space=pl.ANY),
        ],
        out_specs=pl.BlockSpec(memory_space=pltpu.VMEM),
        scratch_shapes=[
            pltpu.VMEM((N_DEV, D_MODEL, D_SHARD), jnp.bfloat16),
            pltpu.VMEM((N_DEV, D_SHARD, D_MODEL), jnp.bfloat16),
            pltpu.VMEM((B_LOC * SQ, D_SHARD), jnp.bfloat16),
            pltpu.VMEM((KV_SLAB, SQ, DH), jnp.bfloat16),
            pltpu.VMEM((KV_SLAB, SQ, DH), jnp.bfloat16),
            pltpu.SemaphoreType.DMA((2,)),
            pltpu.SemaphoreType.DMA((N_DEV - 1,)),
            pltpu.SemaphoreType.DMA((N_DEV - 1,)),
            pltpu.SemaphoreType.DMA((N_DEV - 1,)),
            pltpu.SemaphoreType.DMA((N_DEV - 1,)),
        ],
        compiler_params=pltpu.CompilerParams(collective_id=0),
    )(x2, Wq_bf, Wo_bf, Kt, Vt)
    return out.reshape(B_LOC, SQ, D_MODEL)


# device time: 132067 ns/iter; 1.9320x vs baseline; 1.4001x over previous
import jax
import jax.numpy as jnp
from jax import lax
from jax.experimental import pallas as pl
from jax.experimental.pallas import tpu as pltpu

N_DEV = 8
B_LOC = 2
SQ = 512
HQ_LOC = 8
DH = 64
D_MODEL = 768
D_SHARD = HQ_LOC * DH
KV_SLAB = HQ_LOC * B_LOC
R_HOPS = 4
L_HOPS = 3


def _attn_part(jj, slot, x_ref, kbuf, vbuf, w_buf, ctx_ref, mask):
    wq_t = w_buf[jj, 0]
    qf = lax.dot_general(x_ref[...], wq_t, (((1,), (1,)), ((), ())),
                         preferred_element_type=jnp.float32)
    qf = (qf * 0.125).astype(jnp.bfloat16)
    for b in range(B_LOC):
        for h in range(HQ_LOC):
            q = qf[b * SQ:(b + 1) * SQ, h * DH:(h + 1) * DH]
            k = kbuf[slot, h * B_LOC + b]
            s = lax.dot_general(q, k, (((1,), (1,)), ((), ())),
                                preferred_element_type=jnp.float32)
            w = jnp.exp(jnp.where(mask, s, -1e9))
            denom = jnp.sum(w, axis=1, keepdims=True)
            ctx = jnp.dot(w.astype(jnp.bfloat16), vbuf[slot, h * B_LOC + b],
                          preferred_element_type=jnp.float32)
            ctx_ref[b * SQ:(b + 1) * SQ, h * DH:(h + 1) * DH] = (
                (ctx / denom).astype(jnp.bfloat16))


def kernel(x, Wq, K_ext, V_ext, Wo):
    i_idx = lax.axis_index("i")

    Ks = lax.dynamic_slice_in_dim(K_ext, i_idx * B_LOC, B_LOC, 0)
    Vs = lax.dynamic_slice_in_dim(V_ext, i_idx * B_LOC, B_LOC, 0)
    Kt = jnp.transpose(Ks, (2, 0, 1, 3)).astype(jnp.bfloat16)
    Kt = Kt.reshape(64 * B_LOC, SQ, DH)
    Vt = jnp.transpose(Vs, (2, 0, 1, 3)).astype(jnp.bfloat16)
    Vt = Vt.reshape(64 * B_LOC, SQ, DH)
    x2 = x.reshape(B_LOC * SQ, D_MODEL).astype(jnp.bfloat16)
    Wpack = jnp.stack([Wq.T.astype(jnp.bfloat16), Wo.astype(jnp.bfloat16)])

    def body(x_ref, wpack_ref, kt_hbm, vt_hbm, out_ref,
             w_buf, ctx_ref, kbuf, vbuf, kv_sem,
             sq_r, rq_r, so_r, ro_r, sq_l, rq_l, so_l, ro_l):
        my = lax.axis_index("i")
        left = lax.rem(my + N_DEV - 1, N_DEV)
        right = lax.rem(my + 1, N_DEV)

        barrier_sem = pltpu.get_barrier_semaphore()
        pl.semaphore_signal(barrier_sem, inc=1, device_id=(left,),
                            device_id_type=pl.DeviceIdType.MESH)
        pl.semaphore_signal(barrier_sem, inc=1, device_id=(right,),
                            device_id_type=pl.DeviceIdType.MESH)
        pl.semaphore_wait(barrier_sem, 2)

        w_buf[pl.ds(my, 1)] = wpack_ref[...][None]

        qb = lax.broadcasted_iota(jnp.int32, (SQ, SQ), 0) // 64
        kb = lax.broadcasted_iota(jnp.int32, (SQ, SQ), 1) // 64
        mask = (qb == kb) | (kb == 0) | (lax.rem(qb + kb, 3) == 0)

        out_ref[...] = jnp.zeros_like(out_ref)

        def rdma(half, hop, o_slot, sems_s, sems_r, peer):
            return pltpu.make_async_remote_copy(
                src_ref=w_buf.at[o_slot, half], dst_ref=w_buf.at[o_slot, half],
                send_sem=sems_s.at[hop], recv_sem=sems_r.at[hop],
                device_id=(peer,), device_id_type=pl.DeviceIdType.MESH)

        def start_kv(jj, slot):
            pltpu.make_async_copy(
                kt_hbm.at[pl.ds(jj * KV_SLAB, KV_SLAB)], kbuf.at[slot],
                kv_sem.at[slot, 0]).start()
            pltpu.make_async_copy(
                vt_hbm.at[pl.ds(jj * KV_SLAB, KV_SLAB)], vbuf.at[slot],
                kv_sem.at[slot, 1]).start()

        def wait_kv(slot):
            pltpu.make_async_copy(
                kt_hbm.at[pl.ds(0, KV_SLAB)], kbuf.at[slot],
                kv_sem.at[slot, 0]).wait()
            pltpu.make_async_copy(
                vt_hbm.at[pl.ds(0, KV_SLAB)], vbuf.at[slot],
                kv_sem.at[slot, 1]).wait()

        def proj(jj):
            out_ref[...] += jnp.dot(ctx_ref[...], w_buf[jj, 1],
                                    preferred_element_type=jnp.float32)

        rdma(0, 0, my, sq_r, rq_r, right).start()
        rdma(0, 0, my, sq_l, rq_l, left).start()
        rdma(1, 0, my, so_r, ro_r, right).start()
        rdma(1, 0, my, so_l, ro_l, left).start()
        start_kv(my, 0)
        wait_kv(0)
        _attn_part(my, 0, x_ref, kbuf, vbuf, w_buf, ctx_ref, mask)
        proj(my)

        def round_(r, _):
            o_r = lax.rem(my + N_DEV - 1 - r, N_DEV)
            o_l = lax.rem(my + 1 + r, N_DEV)
            start_kv(o_r, 0)
            start_kv(o_l, 1)
            rdma(0, r, o_r, sq_r, rq_r, left).wait_recv()

            @pl.when(r < R_HOPS - 1)
            def _():
                rdma(0, r + 1, o_r, sq_r, rq_r, right).start()

            rdma(0, r, o_l, sq_l, rq_l, right).wait_recv()

            @pl.when(r < L_HOPS - 1)
            def _():
                rdma(0, r + 1, o_l, sq_l, rq_l, left).start()

            rdma(1, r, o_r, so_r, ro_r, left).wait_recv()

            @pl.when(r < R_HOPS - 1)
            def _():
                rdma(1, r + 1, o_r, so_r, ro_r, right).start()

            rdma(1, r, o_l, so_l, ro_l, right).wait_recv()

            @pl.when(r < L_HOPS - 1)
            def _():
                rdma(1, r + 1, o_l, so_l, ro_l, left).start()

            rdma(0, r, o_r, sq_r, rq_r, right).wait_send()
            rdma(1, r, o_r, so_r, ro_r, right).wait_send()
            rdma(0, r, o_l, sq_l, rq_l, left).wait_send()
            rdma(1, r, o_l, so_l, ro_l, left).wait_send()

            wait_kv(0)
            _attn_part(o_r, 0, x_ref, kbuf, vbuf, w_buf, ctx_ref, mask)
            proj(o_r)
            wait_kv(1)
            _attn_part(o_l, 1, x_ref, kbuf, vbuf, w_buf, ctx_ref, mask)
            proj(o_l)
            return 0

        lax.fori_loop(0, L_HOPS, round_, 0)

        o_last = lax.rem(my + N_DEV - R_HOPS, N_DEV)
        start_kv(o_last, 0)
        rdma(0, R_HOPS - 1, o_last, sq_r, rq_r, left).wait_recv()
        rdma(1, R_HOPS - 1, o_last, so_r, ro_r, left).wait_recv()
        rdma(0, R_HOPS - 1, o_last, sq_r, rq_r, right).wait_send()
        rdma(1, R_HOPS - 1, o_last, so_r, ro_r, right).wait_send()
        wait_kv(0)
        _attn_part(o_last, 0, x_ref, kbuf, vbuf, w_buf, ctx_ref, mask)
        proj(o_last)

    out = pl.pallas_call(
        body,
        out_shape=jax.ShapeDtypeStruct((B_LOC * SQ, D_MODEL), jnp.float32),
        in_specs=[
            pl.BlockSpec(memory_space=pltpu.VMEM),
            pl.BlockSpec(memory_space=pltpu.VMEM),
            pl.BlockSpec(memory_space=pl.ANY),
            pl.BlockSpec(memory_space=pl.ANY),
        ],
        out_specs=pl.BlockSpec(memory_space=pltpu.VMEM),
        scratch_shapes=[
            pltpu.VMEM((N_DEV, 2, D_SHARD, D_MODEL), jnp.bfloat16),
            pltpu.VMEM((B_LOC * SQ, D_SHARD), jnp.bfloat16),
            pltpu.VMEM((2, KV_SLAB, SQ, DH), jnp.bfloat16),
            pltpu.VMEM((2, KV_SLAB, SQ, DH), jnp.bfloat16),
            pltpu.SemaphoreType.DMA((2, 2)),
            pltpu.SemaphoreType.DMA((R_HOPS,)),
            pltpu.SemaphoreType.DMA((R_HOPS,)),
            pltpu.SemaphoreType.DMA((R_HOPS,)),
            pltpu.SemaphoreType.DMA((R_HOPS,)),
            pltpu.SemaphoreType.DMA((L_HOPS,)),
            pltpu.SemaphoreType.DMA((L_HOPS,)),
            pltpu.SemaphoreType.DMA((L_HOPS,)),
            pltpu.SemaphoreType.DMA((L_HOPS,)),
        ],
        compiler_params=pltpu.CompilerParams(collective_id=0),
    )(x2, Wpack, Kt, Vt)
    return out.reshape(B_LOC, SQ, D_MODEL)
